# baseline (speedup 1.0000x reference)
import functools

import jax
import jax.numpy as jnp
from jax import lax
from jax.experimental import pallas as pl
from jax.experimental.pallas import tpu as pltpu
from jax.experimental.pallas import tpu_sc as plsc

VOCAB = 100000
DIM = 64
BATCH = 4
SEQ = 8192
TOTAL = BATCH * SEQ
NW = 32
PER_W = TOTAL // NW
KCH = 128
NCH = PER_W // KCH
HALF = PER_W // 2
LANES = 16
VPR = DIM // LANES

_mesh = plsc.VectorSubcoreMesh(core_axis_name="c", subcore_axis_name="s")


@functools.partial(
    pl.kernel,
    mesh=_mesh,
    compiler_params=pltpu.CompilerParams(use_tc_tiling_on_sc=False),
    out_type=jax.ShapeDtypeStruct((TOTAL, DIM), jnp.float32),
    scratch_types=[
        pltpu.VMEM((PER_W,), jnp.int32),
        pltpu.VMEM((PER_W, DIM), jnp.float32),
        pltpu.VMEM((HALF, DIM), jnp.float32),
        pltpu.SemaphoreType.DMA,
        pltpu.SemaphoreType.DMA,
        pltpu.SemaphoreType.DMA,
    ],
)
def _embed(ids_hbm, tok_hbm, pos_hbm, out_hbm, idx_v, rows_v, pos_v,
           gsem, psem, osem):
    wid = lax.axis_index("s") * 2 + lax.axis_index("c")
    base = wid * PER_W
    pos_base = base % SEQ

    pltpu.sync_copy(ids_hbm.at[pl.ds(base, PER_W)], idx_v)

    gathers = []
    for j in range(NCH):
        gathers.append(
            pltpu.async_copy(
                tok_hbm.at[idx_v.at[pl.ds(j * KCH, KCH)]],
                rows_v.at[pl.ds(j * KCH, KCH)],
                gsem,
            )
        )
    pos0 = pltpu.async_copy(pos_hbm.at[pl.ds(pos_base, HALF)], pos_v, psem)
    for g in gathers:
        g.wait()
    pos0.wait()

    def add_half(r, _):
        for j in range(VPR):
            sl = pl.ds(j * LANES, LANES)
            rows_v[r, sl] = rows_v[r, sl] + pos_v[r, sl]
        return ()

    lax.fori_loop(0, HALF, add_half, (), unroll=2)

    out0 = pltpu.async_copy(
        rows_v.at[pl.ds(0, HALF)], out_hbm.at[pl.ds(base, HALF)], osem)
    pos1 = pltpu.async_copy(
        pos_hbm.at[pl.ds(pos_base + HALF, HALF)], pos_v, psem)
    pos1.wait()

    def add_half2(r, _):
        for j in range(VPR):
            sl = pl.ds(j * LANES, LANES)
            rows_v[HALF + r, sl] = rows_v[HALF + r, sl] + pos_v[r, sl]
        return ()

    lax.fori_loop(0, HALF, add_half2, (), unroll=2)

    out1 = pltpu.async_copy(
        rows_v.at[pl.ds(HALF, HALF)], out_hbm.at[pl.ds(base + HALF, HALF)],
        osem)
    out0.wait()
    out1.wait()


def kernel(input_ids, token_table, pos_table):
    ids = input_ids.reshape(-1).astype(jnp.int32)
    out = _embed(ids, token_table, pos_table)
    return out.reshape(BATCH, SEQ, DIM)
